# baseline (speedup 1.0000x reference)
import functools

import jax
import jax.numpy as jnp
from jax import lax
from jax.experimental import pallas as pl
from jax.experimental.pallas import tpu as pltpu
from jax.experimental.pallas import tpu_sc as plsc

D_MODEL = 64


@functools.lru_cache(maxsize=None)
def _embed_lookup(B: int, C: int = 320):
    info = plsc.get_sparse_core_info()
    NC, NS = info.num_cores, info.num_subcores
    NW = NC * NS
    assert B % (8 * NW) == 0
    b_per_w = B // NW
    assert b_per_w % (4 * C) == 0
    n_chunks = b_per_w // C
    n_quads = n_chunks // 4
    mesh = plsc.VectorSubcoreMesh(core_axis_name="c", subcore_axis_name="s")

    @functools.partial(
        pl.kernel,
        mesh=mesh,
        out_type=jax.ShapeDtypeStruct((B, D_MODEL), jnp.float32),
        scratch_types=[
            pltpu.VMEM((b_per_w,), jnp.int32),
            pltpu.VMEM((4, C, D_MODEL), jnp.float32),
            pltpu.SemaphoreType.DMA,
            pltpu.SemaphoreType.DMA,
            pltpu.SemaphoreType.DMA,
            pltpu.SemaphoreType.DMA,
            pltpu.SemaphoreType.DMA,
            pltpu.SemaphoreType.DMA,
            pltpu.SemaphoreType.DMA,
            pltpu.SemaphoreType.DMA,
        ],
        compiler_params=pltpu.CompilerParams(use_tc_tiling_on_sc=False),
    )
    def body(idx_hbm, table_hbm, out_hbm, idx_v, rows,
             sg0, sg1, sg2, sg3, so0, so1, so2, so3):
        w = lax.axis_index("s") * NC + lax.axis_index("c")
        base = w * b_per_w
        pltpu.sync_copy(idx_hbm.at[pl.ds(base, b_per_w)], idx_v)
        sems_g = (sg0, sg1, sg2, sg3)
        sems_o = (so0, so1, so2, so3)

        def gather(i, b):
            pltpu.async_copy(
                table_hbm.at[idx_v.at[pl.ds(i * C, C)]], rows.at[b], sems_g[b]
            )

        def gather_wait(i, b):
            pltpu.make_async_copy(
                table_hbm.at[idx_v.at[pl.ds(i * C, C)]], rows.at[b], sems_g[b]
            ).wait()

        def flush(i, b):
            pltpu.async_copy(
                rows.at[b], out_hbm.at[pl.ds(base + i * C, C)], sems_o[b]
            )

        def flush_wait(i, b):
            pltpu.make_async_copy(
                rows.at[b], out_hbm.at[pl.ds(base + i * C, C)], sems_o[b]
            ).wait()

        gather(0, 0)
        gather(1, 1)

        def quad(q, carry):
            for b in range(4):
                i = 4 * q + b
                gather_wait(i, b)
                flush(i, b)
                if b < 2:
                    @pl.when(q > 0)
                    def _():
                        flush_wait(i - 2, (b + 2) % 4)
                        gather(i + 2, (b + 2) % 4)

                    @pl.when(q == 0)
                    def _():
                        gather(i + 2, (b + 2) % 4)
                else:
                    flush_wait(i - 2, (b + 2) % 4)

                    @pl.when(q < n_quads - 1)
                    def _():
                        gather(i + 2, (b + 2) % 4)

            return carry

        lax.fori_loop(0, n_quads, quad, 0)
        flush_wait(n_chunks - 2, 2)
        flush_wait(n_chunks - 1, 3)

    return body


def kernel(tokens, W_E):
    n_seq, n_tok = tokens.shape
    B = n_seq * n_tok
    flat = tokens.reshape(B)
    out = _embed_lookup(B)(flat, W_E)
    return out.reshape(n_seq, n_tok, D_MODEL)
